# baseline (speedup 1.0000x reference)
import jax
import jax.numpy as jnp
from jax.experimental import pallas as pl
from jax.experimental.pallas import tpu as pltpu

N = 10000
BR = 40
NB = N // (2 * BR)


def _fused_body(x_ref, a1_ref, a2_ref, w1_ref, b1_ref, g_ref, be_ref,
                wp1_ref, bp1_ref, wp2_ref, bp2_ref, zn_ref, p_ref,
                z1_s, z2_s, z3_s):
    i = pl.program_id(0)

    @pl.when(i == 0)
    def _entry():
        z = jnp.dot(x_ref[...], w1_ref[...],
                    preferred_element_type=jnp.float32)
        z1_s[...] = jnp.maximum(z + b1_ref[...], 0.0)

    @pl.when(i < NB)
    def _prop1():
        z2_s[pl.ds(i * 2 * BR, BR), :] = jnp.dot(
            a1_ref[...], z1_s[...], preferred_element_type=jnp.float32)
        z2_s[pl.ds(i * 2 * BR + BR, BR), :] = jnp.dot(
            a2_ref[...], z1_s[...], preferred_element_type=jnp.float32)

    @pl.when((i >= NB) & (i < 2 * NB))
    def _prop2():
        j = i - NB
        z3_s[pl.ds(j * 2 * BR, BR), :] = jnp.dot(
            a1_ref[...], z2_s[...], preferred_element_type=jnp.float32)
        z3_s[pl.ds(j * 2 * BR + BR, BR), :] = jnp.dot(
            a2_ref[...], z2_s[...], preferred_element_type=jnp.float32)

    @pl.when(i == 2 * NB)
    def _head():
        z = z3_s[...]
        mean = jnp.mean(z, axis=0, keepdims=True)
        var = jnp.mean(jnp.square(z - mean), axis=0, keepdims=True)
        zn = (z - mean) / jnp.sqrt(var + 1e-5) * g_ref[...] + be_ref[...]
        zn_ref[...] = zn
        h = jnp.maximum(
            jnp.dot(zn, wp1_ref[...], preferred_element_type=jnp.float32)
            + bp1_ref[...], 0.0)
        p_ref[...] = (
            jnp.dot(h, wp2_ref[...], preferred_element_type=jnp.float32)
            + bp2_ref[...])


def _a_index(i):
    blk = jnp.where(i < NB, i, jnp.where(i < 2 * NB, i - NB, NB - 1))
    return (blk, 0)


def kernel(x, a, W1, b1, gamma, beta, Wp1, bp1, Wp2, bp2):
    emb = W1.shape[1]
    proj = Wp2.shape[1]
    b1r = b1.reshape(1, emb)
    gr = gamma.reshape(1, emb)
    ber = beta.reshape(1, emb)
    bp1r = bp1.reshape(1, proj)
    bp2r = bp2.reshape(1, proj)

    zn, p = pl.pallas_call(
        _fused_body,
        grid=(2 * NB + 1,),
        in_specs=[
            pl.BlockSpec((N, x.shape[1]), lambda i: (0, 0)),
            pl.BlockSpec((BR, N),
                         lambda i: (2 * _a_index(i)[0], 0)),
            pl.BlockSpec((BR, N),
                         lambda i: (2 * _a_index(i)[0] + 1, 0)),
            pl.BlockSpec((x.shape[1], emb), lambda i: (0, 0)),
            pl.BlockSpec((1, emb), lambda i: (0, 0)),
            pl.BlockSpec((1, emb), lambda i: (0, 0)),
            pl.BlockSpec((1, emb), lambda i: (0, 0)),
            pl.BlockSpec((emb, proj), lambda i: (0, 0)),
            pl.BlockSpec((1, proj), lambda i: (0, 0)),
            pl.BlockSpec((proj, proj), lambda i: (0, 0)),
            pl.BlockSpec((1, proj), lambda i: (0, 0)),
        ],
        out_specs=(
            pl.BlockSpec((N, emb), lambda i: (0, 0)),
            pl.BlockSpec((N, proj), lambda i: (0, 0)),
        ),
        out_shape=(
            jax.ShapeDtypeStruct((N, emb), jnp.float32),
            jax.ShapeDtypeStruct((N, proj), jnp.float32),
        ),
        scratch_shapes=[
            pltpu.VMEM((N, emb), jnp.float32),
            pltpu.VMEM((N, emb), jnp.float32),
            pltpu.VMEM((N, emb), jnp.float32),
        ],
        compiler_params=pltpu.CompilerParams(
            dimension_semantics=("arbitrary",)),
    )(x, a, a, W1, b1r, gr, ber, Wp1, bp1r, Wp2, bp2r)
    return (zn, p)


# speedup vs baseline: 1.4775x; 1.4775x over previous
import jax
import jax.numpy as jnp
from jax.experimental import pallas as pl
from jax.experimental.pallas import tpu as pltpu

N = 10000
BR = 200
NB = N // (2 * BR)
OB = 2000
NBO = N // OB


def _fused_body(x_ref, a1_ref, a2_ref, w1_ref, b1_ref, g_ref, be_ref,
                wp1_ref, bp1_ref, wp2_ref, bp2_ref, zn_ref, p_ref,
                z1_s, z2_s, z3_s, stat_s):
    i = pl.program_id(0)

    @pl.when(i == 0)
    def _entry():
        z = jnp.dot(x_ref[...], w1_ref[...],
                    preferred_element_type=jnp.float32)
        z1_s[...] = jnp.maximum(z + b1_ref[...], 0.0)

    @pl.when(i < NB)
    def _prop1():
        z2_s[pl.ds(i * 2 * BR, BR), :] = jnp.dot(
            a1_ref[...], z1_s[...], preferred_element_type=jnp.float32)
        z2_s[pl.ds(i * 2 * BR + BR, BR), :] = jnp.dot(
            a2_ref[...], z1_s[...], preferred_element_type=jnp.float32)

    @pl.when((i >= NB) & (i < 2 * NB))
    def _prop2():
        j = i - NB
        z3_s[pl.ds(j * 2 * BR, BR), :] = jnp.dot(
            a1_ref[...], z2_s[...], preferred_element_type=jnp.float32)
        z3_s[pl.ds(j * 2 * BR + BR, BR), :] = jnp.dot(
            a2_ref[...], z2_s[...], preferred_element_type=jnp.float32)

    @pl.when(i == 2 * NB)
    def _stats():
        z = z3_s[...]
        mean = jnp.mean(z, axis=0, keepdims=True)
        var = jnp.mean(jnp.square(z - mean), axis=0, keepdims=True)
        stat_s[0:1, :] = mean
        stat_s[1:2, :] = jax.lax.rsqrt(var + 1e-5)

    @pl.when(i >= 2 * NB)
    def _head():
        j = i - 2 * NB
        z = z3_s[pl.ds(j * OB, OB), :]
        zn = ((z - stat_s[0:1, :]) * stat_s[1:2, :] * g_ref[...]
              + be_ref[...])
        zn_ref[...] = zn
        h = jnp.maximum(
            jnp.dot(zn, wp1_ref[...], preferred_element_type=jnp.float32)
            + bp1_ref[...], 0.0)
        p_ref[...] = (
            jnp.dot(h, wp2_ref[...], preferred_element_type=jnp.float32)
            + bp2_ref[...])


def _a_blk(i):
    return jnp.where(i < NB, i, jnp.where(i < 2 * NB, i - NB, NB - 1))


def _out_blk(i):
    return jnp.where(i < 2 * NB, 0, i - 2 * NB)


def kernel(x, a, W1, b1, gamma, beta, Wp1, bp1, Wp2, bp2):
    emb = W1.shape[1]
    proj = Wp2.shape[1]
    b1r = b1.reshape(1, emb)
    gr = gamma.reshape(1, emb)
    ber = beta.reshape(1, emb)
    bp1r = bp1.reshape(1, proj)
    bp2r = bp2.reshape(1, proj)

    zn, p = pl.pallas_call(
        _fused_body,
        grid=(2 * NB + NBO,),
        in_specs=[
            pl.BlockSpec((N, x.shape[1]), lambda i: (0, 0)),
            pl.BlockSpec((BR, N),
                         lambda i: (2 * _a_blk(i), 0)),
            pl.BlockSpec((BR, N),
                         lambda i: (2 * _a_blk(i) + 1, 0)),
            pl.BlockSpec((x.shape[1], emb), lambda i: (0, 0)),
            pl.BlockSpec((1, emb), lambda i: (0, 0)),
            pl.BlockSpec((1, emb), lambda i: (0, 0)),
            pl.BlockSpec((1, emb), lambda i: (0, 0)),
            pl.BlockSpec((emb, proj), lambda i: (0, 0)),
            pl.BlockSpec((1, proj), lambda i: (0, 0)),
            pl.BlockSpec((proj, proj), lambda i: (0, 0)),
            pl.BlockSpec((1, proj), lambda i: (0, 0)),
        ],
        out_specs=(
            pl.BlockSpec((OB, emb), lambda i: (_out_blk(i), 0)),
            pl.BlockSpec((OB, proj), lambda i: (_out_blk(i), 0)),
        ),
        out_shape=(
            jax.ShapeDtypeStruct((N, emb), jnp.float32),
            jax.ShapeDtypeStruct((N, proj), jnp.float32),
        ),
        scratch_shapes=[
            pltpu.VMEM((N, emb), jnp.float32),
            pltpu.VMEM((N, emb), jnp.float32),
            pltpu.VMEM((N, emb), jnp.float32),
            pltpu.VMEM((8, emb), jnp.float32),
        ],
        compiler_params=pltpu.CompilerParams(
            dimension_semantics=("arbitrary",)),
    )(x, a, a, W1, b1r, gr, ber, Wp1, bp1r, Wp2, bp2r)
    return (zn, p)
